# baseline (speedup 1.0000x reference)
import functools

import jax
import jax.numpy as jnp
from jax import lax
from jax.experimental import pallas as pl
from jax.experimental.pallas import tpu as pltpu, tpu_sc as plsc

_B = 16384
_D = 64

_NC = 2
_NS = 16
_NW = _NC * _NS
_BPW = _B // _NW
_G = 16
_SPLIT = 512
_H = 128

_mesh = plsc.VectorSubcoreMesh(core_axis_name="c", subcore_axis_name="s")


@functools.partial(
    pl.kernel,
    mesh=_mesh,
    out_type=(
        jax.ShapeDtypeStruct((_B, _D), jnp.float32),
        jax.ShapeDtypeStruct((_B, _D), jnp.float32),
        jax.ShapeDtypeStruct((_B, _D), jnp.float32),
    ),
    scratch_types=[
        pltpu.VMEM((_BPW,), jnp.int32),
        pltpu.VMEM((_BPW,), jnp.int32),
        pltpu.VMEM((_BPW,), jnp.int32),
        pltpu.VMEM((_BPW,), jnp.int32),
        pltpu.VMEM((_H, 2 * _D), jnp.float32),
        pltpu.VMEM((_BPW, _D), jnp.float32),
        pltpu.SemaphoreType.DMA,
        pltpu.SemaphoreType.DMA,
        pltpu.SemaphoreType.DMA,
        pltpu.SemaphoreType.DMA,
        pltpu.SemaphoreType.DMA,
        pltpu.SemaphoreType.DMA,
        pltpu.SemaphoreType.DMA,
        pltpu.SemaphoreType.DMA,
    ],
    compiler_params=pltpu.CompilerParams(
        disable_bounds_checks=True, disable_semaphore_checks=True
    ),
)
def _lookup(h_hbm, t_hbm, r_hbm, ent_hbm, rel2_hbm,
            h_out, t_out, r_out,
            hi_v, ti_v, ri_v, pair_v, chunk_v, stage_v, *sems):
    wid = lax.axis_index("s") * _NC + lax.axis_index("c")
    base = wid * _BPW
    sl = pl.ds(base, _BPW)
    pltpu.sync_copy(h_hbm.at[sl], hi_v)
    pltpu.sync_copy(t_hbm.at[sl], ti_v)
    pltpu.sync_copy(r_hbm.at[sl], ri_v)
    ssem = sems[:4]
    dsem = sems[4:]

    def gather_ht(idx_v, out):
        def sbody(g, carry):
            iv = idx_v[pl.ds(g * _G, _G)]
            for k in range(_G):
                row = g * _G + k
                pltpu.async_copy(ent_hbm.at[pl.ds(iv[k], 1), :],
                                 stage_v.at[pl.ds(row, 1), :], ssem[k % 4])
            return carry

        lax.fori_loop(0, _SPLIT // _G, sbody, 0)

        def dbody(g, carry):
            off = _SPLIT + g * _G
            iv = idx_v[pl.ds(off, _G)]
            for k in range(_G):
                pltpu.async_copy(ent_hbm.at[pl.ds(iv[k], 1), :],
                                 out.at[pl.ds(base + off + k, 1), :],
                                 dsem[k % 4])
            return carry

        lax.fori_loop(0, (_BPW - _SPLIT) // _G, dbody, 0)

        for s in range(4):
            def sdrain(i, carry):
                pltpu.make_async_copy(ent_hbm.at[pl.ds(0, 1), :],
                                      stage_v.at[pl.ds(0, 1), :],
                                      ssem[s]).wait()
                return carry

            lax.fori_loop(0, _SPLIT // 4, sdrain, 0)
        pltpu.sync_copy(stage_v.at[pl.ds(0, _SPLIT)],
                        out.at[pl.ds(base, _SPLIT)])
        for s in range(4):
            def ddrain(i, carry):
                pltpu.make_async_copy(ent_hbm.at[pl.ds(0, 1), :],
                                      out.at[pl.ds(base, 1), :],
                                      dsem[s]).wait()
                return carry

            lax.fori_loop(0, (_BPW - _SPLIT) // 4, ddrain, 0)

    gather_ht(hi_v, h_out)
    gather_ht(ti_v, t_out)

    def rprep(g, carry):
        iv = ri_v[pl.ds(g * _G, _G)]
        pair_v[pl.ds(g * _G, _G)] = iv >> 1
        return carry

    lax.fori_loop(0, _BPW // _G, rprep, 0)

    def rhalf(hh, carry):
        hoff = hh * _H
        pltpu.async_copy(
            rel2_hbm.at[pair_v.at[pl.ds(hoff, _H)]], chunk_v, ssem[0]
        ).wait()
        for g in range(_H // _G):
            iv = ri_v[pl.ds(hoff + g * _G, _G)]
            for k in range(_G):
                off = (iv[k] & 1) * _D
                row = hoff + g * _G + k
                for c in range(0, _D, 16):
                    stage_v[row, pl.ds(c, 16)] = \
                        chunk_v[g * _G + k, pl.ds(off + c, 16)]
        return carry

    lax.fori_loop(0, _BPW // _H, rhalf, 0)
    pltpu.sync_copy(stage_v, r_out.at[sl])


def kernel(h, t, r, entity_table, relation_table):
    rel2 = relation_table.reshape(500, 2 * _D)
    return _lookup(h.astype(jnp.int32), t.astype(jnp.int32),
                   r.astype(jnp.int32), entity_table, rel2)


# speedup vs baseline: 1.0308x; 1.0308x over previous
import functools

import jax
import jax.numpy as jnp
from jax import lax
from jax.experimental import pallas as pl
from jax.experimental.pallas import tpu as pltpu, tpu_sc as plsc

_B = 16384
_D = 64

_NC = 2
_NS = 16
_NW = _NC * _NS
_BPW = _B // _NW
_G = 16
_NG = _BPW // _G

_mesh = plsc.VectorSubcoreMesh(core_axis_name="c", subcore_axis_name="s")


@functools.partial(
    pl.kernel,
    mesh=_mesh,
    out_type=(
        jax.ShapeDtypeStruct((_B, _D), jnp.float32),
        jax.ShapeDtypeStruct((_B, _D), jnp.float32),
        jax.ShapeDtypeStruct((_B, _D), jnp.float32),
    ),
    scratch_types=[
        pltpu.VMEM((_BPW,), jnp.int32),
        pltpu.VMEM((_BPW,), jnp.int32),
        pltpu.VMEM((_BPW,), jnp.int32),
        pltpu.VMEM((_BPW, _D), jnp.float32),
        pltpu.SemaphoreType.DMA,
    ],
)
def _lookup(h_hbm, t_hbm, r_hbm, ent_hbm, rel_hbm,
            h_out, t_out, r_out,
            hi_v, ti_v, ri_v, stage_v, sem):
    wid = lax.axis_index("s") * _NC + lax.axis_index("c")
    base = wid * _BPW
    sl = pl.ds(base, _BPW)
    pltpu.sync_copy(h_hbm.at[sl], hi_v)
    pltpu.sync_copy(t_hbm.at[sl], ti_v)
    pltpu.sync_copy(r_hbm.at[sl], ri_v)

    def gather_one(idx_v, tbl, out):
        def body(g, carry):
            iv = idx_v[pl.ds(g * _G, _G)]
            for k in range(_G):
                row = g * _G + k
                pltpu.async_copy(tbl.at[pl.ds(iv[k], 1), :],
                                 stage_v.at[pl.ds(row, 1), :], sem)
            return carry

        lax.fori_loop(0, _NG, body, 0)

        def drain(i, carry):
            pltpu.make_async_copy(tbl.at[pl.ds(0, 1), :],
                                  stage_v.at[pl.ds(0, 1), :], sem).wait()
            return carry

        lax.fori_loop(0, _BPW, drain, 0)
        pltpu.sync_copy(stage_v, out.at[sl])

    gather_one(hi_v, ent_hbm, h_out)
    gather_one(ti_v, ent_hbm, t_out)
    gather_one(ri_v, rel_hbm, r_out)


def kernel(h, t, r, entity_table, relation_table):
    return _lookup(h.astype(jnp.int32), t.astype(jnp.int32),
                   r.astype(jnp.int32), entity_table, relation_table)
